# baseline (speedup 1.0000x reference)
import jax
import jax.numpy as jnp
import numpy as np
from jax.experimental import pallas as pl
from jax.experimental.pallas import tpu as pltpu

_P = 512
_STRIDE = 64
_H, _W = 4000, 6000
_NH = (_H - _P) // _STRIDE + 1
_NW = (_W - _P) // _STRIDE + 1
_NB = 63
_PK = 8
_PL = _W // _PK
_NC = 47
_THIRD = np.float32(1.0 / 3.0)


def _scan_rows_kernel(x_ref, v1_ref, v2_ref,
                      c1_ref, c2_ref, e1_ref, e2_ref, seed_ref):
    k = pl.program_id(0)
    x = x_ref[...]
    gray = ((x[0] + x[1]) + x[2]) * _THIRD
    g2 = gray * gray

    even = k % 2 == 0

    @pl.when(even)
    def _():
        seed_ref[0] = jnp.zeros_like(seed_ref[0])
        seed_ref[1] = jnp.zeros_like(seed_ref[1])

    acc1 = seed_ref[0] + gray[0]
    acc2 = seed_ref[1] + g2[0]
    c1_ref[k] = acc1
    c2_ref[k] = acc2
    for i in range(1, 64):
        acc1 = acc1 + gray[i]
        acc2 = acc2 + g2[i]
    e1_ref[k] = acc1
    e2_ref[k] = acc2
    seed_ref[0] = acc1
    seed_ref[1] = acc2

    @pl.when(k == _NB - 1)
    def _finalize():
        car1 = e1_ref[1]
        car2 = e2_ref[1]
        v1_ref[0] = c1_ref[0]
        v2_ref[0] = c2_ref[0]
        v1_ref[1] = c1_ref[1]
        v2_ref[1] = c2_ref[1]
        for m in range(2, _NB):
            j = m // 2
            v1_ref[m] = car1 + c1_ref[m]
            v2_ref[m] = car2 + c2_ref[m]
            if m % 2 == 1 and m < _NB - 1:
                car1 = car1 + e1_ref[2 * j + 1]
                car2 = car2 + e2_ref[2 * j + 1]


def _hscan_select_kernel(vt1_ref, vt2_ref, sc_ref):
    a0_1 = vt1_ref[0]
    a0_2 = vt2_ref[0]
    acc1 = a0_1
    acc2 = a0_2
    a64_1 = a0_1
    a64_2 = a0_2
    for i in range(1, 128):
        acc1 = acc1 + vt1_ref[i]
        acc2 = acc2 + vt2_ref[i]
        if i == 64:
            a64_1 = acc1
            a64_2 = acc2
    tot1 = acc1
    tot2 = acc2

    hc1 = [tot1[:, 0:1]]
    hc2 = [tot2[:, 0:1]]
    for c in range(1, 46):
        hc1.append(hc1[-1] + tot1[:, c:c + 1])
        hc2.append(hc2[-1] + tot2[:, c:c + 1])
    hc1 = jnp.concatenate(hc1, axis=1)
    hc2 = jnp.concatenate(hc2, axis=1)

    ie1 = jnp.concatenate([a0_1[:, 0:1], hc1 + a0_1[:, 1:47]], axis=1)
    io1 = jnp.concatenate([a64_1[:, 0:1], hc1 + a64_1[:, 1:47]], axis=1)
    ie2 = jnp.concatenate([a0_2[:, 0:1], hc2 + a0_2[:, 1:47]], axis=1)
    io2 = jnp.concatenate([a64_2[:, 0:1], hc2 + a64_2[:, 1:47]], axis=1)

    def box(ii):
        br = ii[8:63, 4:47]
        tl = ii[0:55, 0:43]
        tr = ii[0:55, 4:47]
        bl = ii[8:63, 0:43]
        return ((br + tl) - tr) - bl

    n = float(_P * _P)

    def std_of(i1, i2):
        s1 = box(i1)
        s2 = box(i2)
        return jnp.sqrt((s2 - s1 * s1 / n) / n)

    std_e = std_of(ie1, ie2)
    std_o = std_of(io1, io2)

    minval = jnp.minimum(jnp.min(std_e), jnp.min(std_o))
    ih = jax.lax.broadcasted_iota(jnp.int32, (_NH, 43), 0)
    uu = jax.lax.broadcasted_iota(jnp.int32, (_NH, 43), 1)
    big = jnp.int32(2 ** 30)
    lin_e = ih * _NW + 2 * uu
    lin_o = lin_e + 1
    idx = jnp.minimum(
        jnp.min(jnp.where(std_e == minval, lin_e, big)),
        jnp.min(jnp.where(std_o == minval, lin_o, big)))
    hblk = idx // _NW
    wb = idx % _NW
    sc_ref[0] = hblk
    sc_ref[1] = wb // 2
    sc_ref[2] = wb % 2


def _crop_kernel(sc_ref, t0_ref, t1_ref, t2_ref, t3_ref, t4_ref, out_ref):
    par = sc_ref[2]
    c = jnp.concatenate([t0_ref[...], t1_ref[...], t2_ref[...],
                         t3_ref[...], t4_ref[...]], axis=2)
    out_ref[...] = jnp.where(par == 1, c[:, :, 64:576], c[:, :, 0:512])


def kernel(tensor):
    C, H, W = tensor.shape
    f32 = jnp.float32

    tr = tensor.reshape(3, H, _PK, _PL)
    v1p, v2p = pl.pallas_call(
        _scan_rows_kernel,
        grid=(_NB,),
        in_specs=[pl.BlockSpec((3, 64, _PK, _PL), lambda k: (0, k, 0, 0))],
        out_specs=[pl.BlockSpec((_NB, _PK, _PL), lambda k: (0, 0, 0))] * 2,
        out_shape=[jax.ShapeDtypeStruct((_NB, _PK, _PL), f32)] * 2,
        scratch_shapes=[pltpu.VMEM((_NB, _PK, _PL), f32)] * 4
        + [pltpu.VMEM((2, _PK, _PL), f32)],
        compiler_params=pltpu.CompilerParams(
            dimension_semantics=("arbitrary",)),
        name="scan_rows",
    )(tr)

    def to_chunks(vp):
        v = vp.reshape(_NB, W)
        v = jnp.pad(v, ((0, 0), (1, 128 * _NC - W - 1)))
        vt = jnp.transpose(v.reshape(_NB * _NC, 128))
        return vt.reshape(128, _NB, _NC)

    vt1 = to_chunks(v1p)
    vt2 = to_chunks(v2p)

    sc = pl.pallas_call(
        _hscan_select_kernel,
        in_specs=[
            pl.BlockSpec((128, _NB, _NC), lambda: (0, 0, 0)),
            pl.BlockSpec((128, _NB, _NC), lambda: (0, 0, 0)),
        ],
        out_specs=pl.BlockSpec(memory_space=pltpu.SMEM),
        out_shape=jax.ShapeDtypeStruct((3,), jnp.int32),
        name="hscan_select",
    )(vt1, vt2)

    crop = pl.pallas_call(
        _crop_kernel,
        grid_spec=pltpu.PrefetchScalarGridSpec(
            num_scalar_prefetch=1,
            grid=(8,),
            in_specs=[
                pl.BlockSpec((3, 64, 128),
                             lambda i, s, t=t: (0, s[0] + i, s[1] + t))
                for t in range(5)
            ],
            out_specs=pl.BlockSpec((3, 64, _P), lambda i, s: (0, i, 0)),
        ),
        out_shape=jax.ShapeDtypeStruct((C, _P, _P), f32),
        name="crop",
    )(sc, tensor, tensor, tensor, tensor, tensor)

    return crop


# speedup vs baseline: 1.0069x; 1.0069x over previous
import jax
import jax.numpy as jnp
import numpy as np
from jax.experimental import pallas as pl
from jax.experimental.pallas import tpu as pltpu

_P = 512
_STRIDE = 64
_H, _W = 4000, 6000
_NH = (_H - _P) // _STRIDE + 1
_NW = (_W - _P) // _STRIDE + 1
_NB = 63
_PK = 8
_PL = _W // _PK
_NC = 47
_THIRD = np.float32(1.0 / 3.0)


def _scan_rows_kernel(x_ref, v1_ref, v2_ref,
                      c1_ref, c2_ref, e1_ref, e2_ref, seed_ref):
    k = pl.program_id(0)
    x = x_ref[...]
    gray = ((x[0] + x[1]) + x[2]) * _THIRD
    g2 = gray * gray

    even = k % 2 == 0

    @pl.when(even)
    def _():
        seed_ref[0] = jnp.zeros_like(seed_ref[0])
        seed_ref[1] = jnp.zeros_like(seed_ref[1])

    acc1 = seed_ref[0] + gray[0]
    acc2 = seed_ref[1] + g2[0]
    c1_ref[k] = acc1
    c2_ref[k] = acc2
    for i in range(1, 64):
        acc1 = acc1 + gray[i]
        acc2 = acc2 + g2[i]
    e1_ref[k] = acc1
    e2_ref[k] = acc2
    seed_ref[0] = acc1
    seed_ref[1] = acc2

    @pl.when(k == _NB - 1)
    def _finalize():
        car1 = e1_ref[1]
        car2 = e2_ref[1]
        v1_ref[0] = c1_ref[0]
        v2_ref[0] = c2_ref[0]
        v1_ref[1] = c1_ref[1]
        v2_ref[1] = c2_ref[1]
        for m in range(2, _NB):
            j = m // 2
            v1_ref[m] = car1 + c1_ref[m]
            v2_ref[m] = car2 + c2_ref[m]
            if m % 2 == 1 and m < _NB - 1:
                car1 = car1 + e1_ref[2 * j + 1]
                car2 = car2 + e2_ref[2 * j + 1]


def _hscan_select_kernel(vt1_ref, vt2_ref, sc_ref):
    a0_1 = vt1_ref[0]
    a0_2 = vt2_ref[0]
    acc1 = a0_1
    acc2 = a0_2
    a64_1 = a0_1
    a64_2 = a0_2
    for i in range(1, 128):
        acc1 = acc1 + vt1_ref[i]
        acc2 = acc2 + vt2_ref[i]
        if i == 64:
            a64_1 = acc1
            a64_2 = acc2
    tot1 = acc1
    tot2 = acc2

    hc1 = [tot1[:, 0:1]]
    hc2 = [tot2[:, 0:1]]
    for c in range(1, 46):
        hc1.append(hc1[-1] + tot1[:, c:c + 1])
        hc2.append(hc2[-1] + tot2[:, c:c + 1])
    hc1 = jnp.concatenate(hc1, axis=1)
    hc2 = jnp.concatenate(hc2, axis=1)

    ie1 = jnp.concatenate([a0_1[:, 0:1], hc1 + a0_1[:, 1:47]], axis=1)
    io1 = jnp.concatenate([a64_1[:, 0:1], hc1 + a64_1[:, 1:47]], axis=1)
    ie2 = jnp.concatenate([a0_2[:, 0:1], hc2 + a0_2[:, 1:47]], axis=1)
    io2 = jnp.concatenate([a64_2[:, 0:1], hc2 + a64_2[:, 1:47]], axis=1)

    def box(ii):
        br = ii[8:63, 4:47]
        tl = ii[0:55, 0:43]
        tr = ii[0:55, 4:47]
        bl = ii[8:63, 0:43]
        return ((br + tl) - tr) - bl

    n = float(_P * _P)

    def std_of(i1, i2):
        s1 = box(i1)
        s2 = box(i2)
        return jnp.sqrt((s2 - s1 * s1 / n) / n)

    std_e = std_of(ie1, ie2)
    std_o = std_of(io1, io2)

    minval = jnp.minimum(jnp.min(std_e), jnp.min(std_o))
    ih = jax.lax.broadcasted_iota(jnp.int32, (_NH, 43), 0)
    uu = jax.lax.broadcasted_iota(jnp.int32, (_NH, 43), 1)
    big = jnp.int32(2 ** 30)
    lin_e = ih * _NW + 2 * uu
    lin_o = lin_e + 1
    idx = jnp.minimum(
        jnp.min(jnp.where(std_e == minval, lin_e, big)),
        jnp.min(jnp.where(std_o == minval, lin_o, big)))
    hblk = idx // _NW
    wb = idx % _NW
    sc_ref[0] = hblk
    sc_ref[1] = wb // 2
    sc_ref[2] = wb % 2


def _crop_kernel(sc_ref, t0_ref, t1_ref, t2_ref, t3_ref, t4_ref, out_ref):
    par = sc_ref[2]
    c = jnp.concatenate([t0_ref[...], t1_ref[...], t2_ref[...],
                         t3_ref[...], t4_ref[...]], axis=2)
    out_ref[...] = jnp.where(par == 1, c[:, :, 64:576], c[:, :, 0:512])


def kernel(tensor):
    C, H, W = tensor.shape
    f32 = jnp.float32

    tr = tensor.reshape(3, H, _PK, _PL)
    v1p, v2p = pl.pallas_call(
        _scan_rows_kernel,
        grid=(_NB,),
        in_specs=[pl.BlockSpec((3, 64, _PK, _PL), lambda k: (0, k, 0, 0))],
        out_specs=[pl.BlockSpec((_NB, _PK, _PL), lambda k: (0, 0, 0))] * 2,
        out_shape=[jax.ShapeDtypeStruct((_NB, _PK, _PL), f32)] * 2,
        scratch_shapes=[pltpu.VMEM((_NB, _PK, _PL), f32)] * 4
        + [pltpu.VMEM((2, _PK, _PL), f32)],
        compiler_params=pltpu.CompilerParams(
            dimension_semantics=("arbitrary",)),
        name="scan_rows",
    )(tr)

    def to_chunks(vp):
        v = vp.reshape(_NB, W)
        v = jnp.pad(v, ((0, 0), (1, 128 * _NC - W - 1)))
        return jnp.transpose(v.reshape(_NB, _NC, 128), (2, 0, 1))

    vt1 = to_chunks(v1p)
    vt2 = to_chunks(v2p)

    sc = pl.pallas_call(
        _hscan_select_kernel,
        in_specs=[
            pl.BlockSpec((128, _NB, _NC), lambda: (0, 0, 0)),
            pl.BlockSpec((128, _NB, _NC), lambda: (0, 0, 0)),
        ],
        out_specs=pl.BlockSpec(memory_space=pltpu.SMEM),
        out_shape=jax.ShapeDtypeStruct((3,), jnp.int32),
        name="hscan_select",
    )(vt1, vt2)

    crop = pl.pallas_call(
        _crop_kernel,
        grid_spec=pltpu.PrefetchScalarGridSpec(
            num_scalar_prefetch=1,
            grid=(8,),
            in_specs=[
                pl.BlockSpec((3, 64, 128),
                             lambda i, s, t=t: (0, s[0] + i, s[1] + t))
                for t in range(5)
            ],
            out_specs=pl.BlockSpec((3, 64, _P), lambda i, s: (0, i, 0)),
        ),
        out_shape=jax.ShapeDtypeStruct((C, _P, _P), f32),
        name="crop",
    )(sc, tensor, tensor, tensor, tensor, tensor)

    return crop
